# Initial kernel scaffold; baseline (speedup 1.0000x reference)
import jax
import jax.numpy as jnp
from jax.experimental import pallas as pl


def kernel(x, edge_index, W1, attn_l1, attn_r1, b1, W2, attn_l2, attn_r2, b2):
    raise NotImplementedError("write your pallas kernel here")


# speedup vs baseline: 12.7758x; 12.7758x over previous
import jax
import jax.numpy as jnp
from jax import lax
from jax.experimental import pallas as pl
from jax.experimental.pallas import tpu as pltpu
from jax.experimental.pallas import tpu_sc as plsc

N = 10000
IN_DIM = 128
H1 = 8
HID = 16
OUT = 16
D1 = H1 * HID

NP = 10240
E_IN = 320000
EREAL = E_IN + N
NW = 32
KC = 128
GPW = 21
CPW = 4 * GPW
EPT = CPW * KC
EP = NW * EPT
NR = EP // KC
NE8 = EP // 8
BR = 1280
GRID = NP // BR
RPS = NP // 16

_mesh = plsc.VectorSubcoreMesh(core_axis_name="c", subcore_axis_name="s")


def _a1_body(x_ref, w_ref, al_ref, ar_ref, z_ref, el_ref, er_ref):
  z = jnp.dot(x_ref[...], w_ref[...], preferred_element_type=jnp.float32)
  z_ref[...] = z
  el16 = jnp.dot(z, al_ref[...], preferred_element_type=jnp.float32)
  el_ref[...] = jnp.concatenate(
      [el16, jnp.zeros((BR, D1 - 16), jnp.float32)], axis=1)
  er16 = jnp.dot(z, ar_ref[...], preferred_element_type=jnp.float32)
  er_ref[...] = jnp.concatenate(
      [er16, jnp.zeros((BR, D1 - 16), jnp.float32)], axis=1)


def _a1(xp, w1, al16, ar16):
  return pl.pallas_call(
      _a1_body,
      grid=(GRID,),
      in_specs=[
          pl.BlockSpec((BR, IN_DIM), lambda i: (i, 0)),
          pl.BlockSpec((IN_DIM, D1), lambda i: (0, 0)),
          pl.BlockSpec((D1, 16), lambda i: (0, 0)),
          pl.BlockSpec((D1, 16), lambda i: (0, 0)),
      ],
      out_specs=[
          pl.BlockSpec((BR, D1), lambda i: (i, 0)),
          pl.BlockSpec((BR, D1), lambda i: (i, 0)),
          pl.BlockSpec((BR, D1), lambda i: (i, 0)),
      ],
      out_shape=[
          jax.ShapeDtypeStruct((NP, D1), jnp.float32),
          jax.ShapeDtypeStruct((NP, D1), jnp.float32),
          jax.ShapeDtypeStruct((NP, D1), jnp.float32),
      ],
  )(xp, w1, al16, ar16)


def _a3_body(n_ref, d_ref, b_ref, w2_ref, al2_ref, ar2_ref,
             zel_ref, er128_ref):
  num = n_ref[0] + n_ref[1]
  den = d_ref[0] + d_ref[1]
  dinv = 1.0 / (den[:, :H1] + 1e-10)
  h = (num.reshape(BR, H1, HID) * dinv[:, :, None]).reshape(BR, D1)
  h = h + b_ref[...]
  h = jnp.where(h > 0, h, jnp.exp(jnp.minimum(h, 0.0)) - 1.0)
  z2 = jnp.dot(h, w2_ref[...], preferred_element_type=jnp.float32)
  el2 = jnp.sum(z2 * al2_ref[...], axis=1, keepdims=True)
  er2 = jnp.sum(z2 * ar2_ref[...], axis=1, keepdims=True)
  el2b = jnp.broadcast_to(el2, (BR, 16))
  er2b = jnp.broadcast_to(er2, (BR, 16))
  zel_ref[...] = jnp.concatenate(
      [z2, el2b, jnp.zeros((BR, 96), jnp.float32)], axis=1)
  er128_ref[...] = jnp.concatenate(
      [er2b, jnp.zeros((BR, 112), jnp.float32)], axis=1)


def _a3(partsN, partsD, b1, w2, al2, ar2):
  return pl.pallas_call(
      _a3_body,
      grid=(GRID,),
      in_specs=[
          pl.BlockSpec((2, BR, D1), lambda i: (0, i, 0)),
          pl.BlockSpec((2, BR, 128), lambda i: (0, i, 0)),
          pl.BlockSpec((1, D1), lambda i: (0, 0)),
          pl.BlockSpec((D1, OUT), lambda i: (0, 0)),
          pl.BlockSpec((1, OUT), lambda i: (0, 0)),
          pl.BlockSpec((1, OUT), lambda i: (0, 0)),
      ],
      out_specs=[
          pl.BlockSpec((BR, 128), lambda i: (i, 0)),
          pl.BlockSpec((BR, 128), lambda i: (i, 0)),
      ],
      out_shape=[
          jax.ShapeDtypeStruct((NP, 128), jnp.float32),
          jax.ShapeDtypeStruct((NP, 128), jnp.float32),
      ],
  )(partsN, partsD, b1, w2, al2, ar2)


def _a5_body(nd_ref, b_ref, o_ref):
  nd = nd_ref[0] + nd_ref[1]
  num = nd[:, :OUT]
  den = nd[:, 16:17]
  dinv = 1.0 / (den + 1e-10)
  o_ref[...] = num * dinv + b_ref[...]


def _a5(partsND, b2):
  return pl.pallas_call(
      _a5_body,
      grid=(GRID,),
      in_specs=[
          pl.BlockSpec((2, BR, 128), lambda i: (0, i, 0)),
          pl.BlockSpec((1, OUT), lambda i: (0, 0)),
      ],
      out_specs=pl.BlockSpec((BR, OUT), lambda i: (i, 0)),
      out_shape=jax.ShapeDtypeStruct((NP, OUT), jnp.float32),
  )(partsND, b2)


def _zero_rows(zero_v, acc, s):
  @pl.loop(0, RPS // 16)
  def _z1(i):
    pltpu.sync_copy(zero_v, acc.at[pl.ds(s * RPS + i * 16, 16)])


def _e1d_kernel(elA, erA, sdR, outD, eeO,
                sd8_v, el_v, er_v, ee_v, zero_v, accD):
  c = lax.axis_index("c")
  s = lax.axis_index("s")
  wid = s * 2 + c

  @pl.loop(0, 16)
  def _z0(r):
    for j in range(8):
      zero_v[r, pl.ds(j * 16, 16)] = jnp.zeros((16,), jnp.float32)

  _zero_rows(zero_v, accD, s)
  plsc.subcore_barrier()

  @pl.loop(0, GPW)
  def _g(g):
    gk = wid * GPW + g
    pltpu.sync_copy(sdR.at[pl.ds(8 * gk, 8)], sd8_v)
    for t in range(4):
      ck = gk * 4 + t
      pltpu.sync_copy(elA.at[sd8_v.at[2 * t]], el_v)
      pltpu.sync_copy(erA.at[sd8_v.at[2 * t + 1]], er_v)

      @pl.loop(0, KC // 8)
      def _q(q):
        for j8 in range(8):
          r = q * 8 + j8
          sv = el_v[r, pl.ds(0, 16)] + er_v[r, pl.ds(0, 16)]
          ee = jnp.exp(jnp.maximum(sv, 0.2 * sv))
          ee_v[q, pl.ds(j8 * 16, 16)] = ee
          er_v[r, pl.ds(0, 16)] = ee

      pltpu.sync_copy(ee_v, eeO.at[pl.ds(ck * (KC // 8), 16)])
      pltpu.sync_copy(er_v, accD.at[sd8_v.at[2 * t + 1]], add=True)

  plsc.subcore_barrier()
  pltpu.sync_copy(accD.at[pl.ds(s * RPS, RPS)],
                  outD.at[c, pl.ds(s * RPS, RPS)])


def _e1d(elA, erA, sdR):
  k = pl.kernel(
      _e1d_kernel,
      out_type=(
          jax.ShapeDtypeStruct((2, NP, 128), jnp.float32),
          jax.ShapeDtypeStruct((NE8, 128), jnp.float32),
      ),
      mesh=_mesh,
      scratch_types=[
          pltpu.VMEM((8, KC), jnp.int32),
          pltpu.VMEM((KC, D1), jnp.float32),
          pltpu.VMEM((KC, D1), jnp.float32),
          pltpu.VMEM((16, 128), jnp.float32),
          pltpu.VMEM((16, 128), jnp.float32),
          pltpu.VMEM_SHARED((NP, 128), jnp.float32),
      ],
  )
  return k(elA, erA, sdR)


def _e1n_kernel(zA, eeR, sdR, outN,
                sd8_v, z_v, ee_v, zero_v, accN):
  c = lax.axis_index("c")
  s = lax.axis_index("s")
  wid = s * 2 + c

  @pl.loop(0, 16)
  def _z0(r):
    for j in range(D1 // 16):
      zero_v[r, pl.ds(j * 16, 16)] = jnp.zeros((16,), jnp.float32)

  _zero_rows(zero_v, accN, s)
  plsc.subcore_barrier()

  @pl.loop(0, GPW)
  def _g(g):
    gk = wid * GPW + g
    pltpu.sync_copy(sdR.at[pl.ds(8 * gk, 8)], sd8_v)
    for t in range(4):
      ck = gk * 4 + t
      pltpu.sync_copy(eeR.at[pl.ds(ck * (KC // 8), 16)], ee_v)
      pltpu.sync_copy(zA.at[sd8_v.at[2 * t]], z_v)

      @pl.loop(0, KC // 8)
      def _q(q):
        for j8 in range(8):
          r = q * 8 + j8
          ee = ee_v[q, pl.ds(j8 * 16, 16)]
          for h in range(H1):
            z_v[r, pl.ds(h * HID, 16)] = z_v[r, pl.ds(h * HID, 16)] * ee[h]

      pltpu.sync_copy(z_v, accN.at[sd8_v.at[2 * t + 1]], add=True)

  plsc.subcore_barrier()
  pltpu.sync_copy(accN.at[pl.ds(s * RPS, RPS)],
                  outN.at[c, pl.ds(s * RPS, RPS)])


def _e1n(zA, eeR, sdR):
  k = pl.kernel(
      _e1n_kernel,
      out_type=jax.ShapeDtypeStruct((2, NP, D1), jnp.float32),
      mesh=_mesh,
      scratch_types=[
          pltpu.VMEM((8, KC), jnp.int32),
          pltpu.VMEM((KC, D1), jnp.float32),
          pltpu.VMEM((16, 128), jnp.float32),
          pltpu.VMEM((16, D1), jnp.float32),
          pltpu.VMEM_SHARED((NP, D1), jnp.float32),
      ],
  )
  return k(zA, eeR, sdR)


def _e2_kernel(zelA, er128A, sdR, outND,
               sd8_v, g_v, er_v, zero_v, accND):
  c = lax.axis_index("c")
  s = lax.axis_index("s")
  wid = s * 2 + c

  @pl.loop(0, 16)
  def _z0(r):
    for j in range(8):
      zero_v[r, pl.ds(j * 16, 16)] = jnp.zeros((16,), jnp.float32)

  _zero_rows(zero_v, accND, s)
  plsc.subcore_barrier()

  @pl.loop(0, GPW)
  def _g(g):
    gk = wid * GPW + g
    pltpu.sync_copy(sdR.at[pl.ds(8 * gk, 8)], sd8_v)
    for t in range(4):
      pltpu.sync_copy(zelA.at[sd8_v.at[2 * t]], g_v)
      pltpu.sync_copy(er128A.at[sd8_v.at[2 * t + 1]], er_v)

      @pl.loop(0, KC)
      def _edge(r):
        sv = g_v[r, pl.ds(16, 16)] + er_v[r, pl.ds(0, 16)]
        ee = jnp.exp(jnp.maximum(sv, 0.2 * sv))
        er_v[r, pl.ds(0, 16)] = g_v[r, pl.ds(0, 16)] * ee
        er_v[r, pl.ds(16, 16)] = ee

      pltpu.sync_copy(er_v, accND.at[sd8_v.at[2 * t + 1]], add=True)

  plsc.subcore_barrier()
  pltpu.sync_copy(accND.at[pl.ds(s * RPS, RPS)],
                  outND.at[c, pl.ds(s * RPS, RPS)])


def _e2(zelA, er128A, sdR):
  k = pl.kernel(
      _e2_kernel,
      out_type=jax.ShapeDtypeStruct((2, NP, 128), jnp.float32),
      mesh=_mesh,
      scratch_types=[
          pltpu.VMEM((8, KC), jnp.int32),
          pltpu.VMEM((KC, 128), jnp.float32),
          pltpu.VMEM((KC, 128), jnp.float32),
          pltpu.VMEM((16, 128), jnp.float32),
          pltpu.VMEM_SHARED((NP, 128), jnp.float32),
      ],
  )
  return k(zelA, er128A, sdR)


def kernel(x, edge_index, W1, attn_l1, attn_r1, b1, W2, attn_l2, attn_r2, b2):
  f32 = jnp.float32
  xp = jnp.pad(x.astype(f32), ((0, NP - N), (0, 0)))
  sl = jnp.arange(N, dtype=jnp.int32)
  pad_e = EP - EREAL
  srcR = jnp.concatenate([edge_index[0], sl,
                          jnp.zeros((pad_e,), jnp.int32)]).reshape(NR, 1, KC)
  dstR = jnp.concatenate([edge_index[1], sl,
                          jnp.full((pad_e,), N, jnp.int32)]).reshape(NR, 1, KC)
  sdR = jnp.concatenate([srcR, dstR], axis=1).reshape(2 * NR, KC)

  eye1 = jnp.eye(H1, dtype=f32)
  al16 = jnp.pad((eye1[:, None, :] * attn_l1[:, :, None]).reshape(D1, H1),
                 ((0, 0), (0, 8)))
  ar16 = jnp.pad((eye1[:, None, :] * attn_r1[:, :, None]).reshape(D1, H1),
                 ((0, 0), (0, 8)))
  al2 = attn_l2.reshape(1, OUT)
  ar2 = attn_r2.reshape(1, OUT)

  zA, elA, erA = _a1(xp, W1, al16, ar16)
  partsD1, eeR = _e1d(elA, erA, sdR)
  partsN1 = _e1n(zA, eeR, sdR)
  zelA, er128A = _a3(partsN1, partsD1, b1.reshape(1, D1), W2, al2, ar2)
  partsND2 = _e2(zelA, er128A, sdR)
  outF = _a5(partsND2, b2.reshape(1, OUT))
  return outF[:N]
